# baseline (speedup 1.0000x reference)
import functools

import jax
import jax.numpy as jnp
from jax import lax
from jax.experimental import pallas as pl
from jax.experimental.pallas import tpu as pltpu
from jax.experimental.pallas import tpu_sc as plsc

N_NODES = 10000
F = 128
NC = 2
NS = 16
NW = NC * NS
CHUNK = 128
NPAD = 10240
RPT = NPAD // NS
E_EDGES = 320000
CPT = 80
PHASES = 2
CPP = CPT // PHASES
EPAD = NW * CHUNK * CPT

_mesh = plsc.VectorSubcoreMesh(core_axis_name="c", subcore_axis_name="s")


@functools.partial(
    pl.kernel,
    mesh=_mesh,
    out_type=jax.ShapeDtypeStruct((NC, NPAD, F), jnp.float32),
    scratch_types=[
        pltpu.VMEM((CPT, CHUNK), jnp.int32),
        pltpu.VMEM((CHUNK, F), jnp.float32),
        pltpu.VMEM_SHARED((NPAD, F), jnp.float32),
    ],
)
def _deg_kernel(dst_hbm, ones_hbm, zeros_hbm, out_hbm, idx_v, ones_v, acc_sh):
    c = lax.axis_index("c")
    s = lax.axis_index("s")
    w = c * NS + s
    pltpu.sync_copy(ones_hbm, ones_v)
    pltpu.sync_copy(zeros_hbm.at[pl.ds(s * RPT, RPT)],
                    acc_sh.at[pl.ds(s * RPT, RPT)])
    pltpu.sync_copy(dst_hbm.at[w], idx_v)
    plsc.subcore_barrier()

    def body(j, carry):
        pltpu.sync_copy(ones_v, acc_sh.at[idx_v.at[j]], add=True)
        return carry

    lax.fori_loop(0, CPT, body, 0)
    plsc.subcore_barrier()
    pltpu.sync_copy(acc_sh.at[pl.ds(s * RPT, RPT)],
                    out_hbm.at[c, pl.ds(s * RPT, RPT)])


@functools.partial(
    pl.kernel,
    mesh=_mesh,
    out_type=jax.ShapeDtypeStruct((NC, NPAD, F), jnp.float32),
    scratch_types=[
        pltpu.VMEM((CPP + 8, CHUNK), jnp.int32),
        pltpu.VMEM((CPP, CHUNK), jnp.int32),
        pltpu.VMEM((CHUNK, F), jnp.float32),
        pltpu.VMEM((CHUNK, F), jnp.float32),
        pltpu.VMEM_SHARED((NPAD, F), jnp.float32),
        pltpu.SemaphoreType.DMA,
        pltpu.SemaphoreType.DMA,
    ],
)
def _agg_kernel(h_hbm, src_hbm, dst_hbm, out_hbm, srcv, dstv, buf0, buf1,
                acc_sh, sem0, sem1):
    c = lax.axis_index("c")
    s = lax.axis_index("s")
    w = c * NS + s
    pltpu.sync_copy(h_hbm.at[pl.ds(s * RPT, RPT)], acc_sh.at[pl.ds(s * RPT, RPT)])
    plsc.subcore_barrier()

    bufs = (buf0, buf1)
    sems = (sem0, sem1)
    for ph in range(PHASES):
        pltpu.sync_copy(src_hbm.at[w, pl.ds(ph * CPP, CPP + 8)], srcv)
        pltpu.sync_copy(dst_hbm.at[w, pl.ds(ph * CPP, CPP)], dstv)
        pltpu.async_copy(h_hbm.at[srcv.at[0]], buf0, sem0)

        def body(it, carry):
            jj = it * 2
            for b in range(2):
                j = jj + b
                nxt = 1 - b
                pltpu.make_async_copy(h_hbm.at[pl.ds(0, CHUNK)], bufs[b],
                                      sems[b]).wait()
                pltpu.async_copy(h_hbm.at[srcv.at[j + 1]], bufs[nxt], sems[nxt])
                pltpu.sync_copy(bufs[b], acc_sh.at[dstv.at[j]], add=True)
            return carry

        lax.fori_loop(0, CPP // 2, body, 0)
        pltpu.make_async_copy(h_hbm.at[pl.ds(0, CHUNK)], buf0, sem0).wait()
    plsc.subcore_barrier()
    pltpu.sync_copy(acc_sh.at[pl.ds(s * RPT, RPT)],
                    out_hbm.at[c, pl.ds(s * RPT, RPT)])


def _mm1_body(x_ref, w_ref, degp_ref, out_ref, dinv_ref):
    deg = degp_ref[0, :, 0:1] + degp_ref[1, :, 0:1] + 1.0
    dinv = lax.rsqrt(jnp.maximum(deg, 1.0))
    dinv_ref[...] = jnp.broadcast_to(dinv, dinv_ref.shape)
    out_ref[...] = jnp.dot(x_ref[...], w_ref[...],
                           preferred_element_type=jnp.float32) * dinv


def _mm2_body(p_ref, h_ref, dinv8_ref, w_ref, b_ref, out_ref):
    dinv = dinv8_ref[:, 0:1]
    agg = p_ref[0] + p_ref[1] - h_ref[...]
    x2 = jnp.maximum(agg * dinv + b_ref[0:1, :], 0.0)
    out_ref[...] = jnp.dot(x2, w_ref[...],
                           preferred_element_type=jnp.float32) * dinv


def _fin_body(q_ref, h_ref, dinv8_ref, b_ref, out_ref):
    dinv = dinv8_ref[:, 0:1]
    out_ref[...] = (q_ref[0] + q_ref[1] - h_ref[...]) * dinv + b_ref[0:1, :]


_B = 1024
_GRID = (NPAD // _B,)
_bs_rows = pl.BlockSpec((_B, F), lambda i: (i, 0))
_bs_w = pl.BlockSpec((F, F), lambda i: (0, 0))
_bs_deg = pl.BlockSpec((NC, _B, F), lambda i: (0, i, 0))
_bs_part = pl.BlockSpec((NC, _B, F), lambda i: (0, i, 0))
_bs_b = pl.BlockSpec((8, F), lambda i: (0, 0))
_bs_dinv = pl.BlockSpec((_B, 8), lambda i: (i, 0))

_mm1 = pl.pallas_call(
    _mm1_body, grid=_GRID,
    in_specs=[_bs_rows, _bs_w, _bs_deg],
    out_specs=[_bs_rows, _bs_dinv],
    out_shape=[jax.ShapeDtypeStruct((NPAD, F), jnp.float32),
               jax.ShapeDtypeStruct((NPAD, 8), jnp.float32)],
)

_mm2 = pl.pallas_call(
    _mm2_body, grid=_GRID,
    in_specs=[_bs_part, _bs_rows, _bs_dinv, _bs_w, _bs_b],
    out_specs=_bs_rows,
    out_shape=jax.ShapeDtypeStruct((NPAD, F), jnp.float32),
)

_fin = pl.pallas_call(
    _fin_body, grid=_GRID,
    in_specs=[_bs_part, _bs_rows, _bs_dinv, _bs_b],
    out_specs=_bs_rows,
    out_shape=jax.ShapeDtypeStruct((NPAD, F), jnp.float32),
)


def kernel(x, edge_index, W1, b1, W2, b2):
    src = edge_index[0].astype(jnp.int32)
    dst = edge_index[1].astype(jnp.int32)
    npad_e = EPAD - src.shape[0]
    src_p = jnp.concatenate([src, jnp.zeros((npad_e,), jnp.int32)]) \
        .reshape(NW, CPT, CHUNK)
    src_p = jnp.concatenate([src_p, jnp.zeros((NW, 8, CHUNK), jnp.int32)], axis=1)
    dst_p = jnp.concatenate([dst, jnp.full((npad_e,), N_NODES, jnp.int32)]) \
        .reshape(NW, CPT, CHUNK)
    x_p = jnp.zeros((NPAD, F), jnp.float32).at[:N_NODES].set(x)
    b1_p = jnp.zeros((8, F), jnp.float32).at[0].set(b1)
    b2_p = jnp.zeros((8, F), jnp.float32).at[0].set(b2)

    ones_c = jnp.ones((CHUNK, F), jnp.float32)
    zeros_c = jnp.zeros((NPAD, F), jnp.float32)
    degp = _deg_kernel(dst_p, ones_c, zeros_c)
    h1s, dinv8 = _mm1(x_p, W1, degp)
    p = _agg_kernel(h1s, src_p, dst_p)
    h2s = _mm2(p, h1s, dinv8, W2, b1_p)
    q = _agg_kernel(h2s, src_p, dst_p)
    out = _fin(q, h2s, dinv8, b2_p)
    return out[:N_NODES]


# speedup vs baseline: 1.2088x; 1.2088x over previous
import functools

import jax
import jax.numpy as jnp
from jax import lax
from jax.experimental import pallas as pl
from jax.experimental.pallas import tpu as pltpu
from jax.experimental.pallas import tpu_sc as plsc

N_NODES = 10000
F = 128
NC = 2
NS = 16
NW = NC * NS
CHUNK = 128
NPAD = 10240
RPT = NPAD // NS
E_EDGES = 320000
CPT = 80
PHASES = 2
CPP = CPT // PHASES
EPAD = NW * CHUNK * CPT

_mesh = plsc.VectorSubcoreMesh(core_axis_name="c", subcore_axis_name="s")


@functools.partial(
    pl.kernel,
    mesh=_mesh,
    out_type=jax.ShapeDtypeStruct((NC, NPAD, F), jnp.float32),
    scratch_types=[
        pltpu.VMEM((CPT, CHUNK), jnp.int32),
        pltpu.VMEM((CHUNK, F), jnp.float32),
        pltpu.VMEM_SHARED((NPAD, F), jnp.float32),
    ],
)
def _deg_kernel(dst_hbm, ones_hbm, zeros_hbm, out_hbm, idx_v, ones_v, acc_sh):
    c = lax.axis_index("c")
    s = lax.axis_index("s")
    w = c * NS + s
    pltpu.sync_copy(ones_hbm, ones_v)
    pltpu.sync_copy(zeros_hbm.at[pl.ds(s * RPT, RPT)],
                    acc_sh.at[pl.ds(s * RPT, RPT)])
    pltpu.sync_copy(dst_hbm.at[w], idx_v)
    plsc.subcore_barrier()

    def body(j, carry):
        pltpu.sync_copy(ones_v, acc_sh.at[idx_v.at[j]], add=True)
        return carry

    lax.fori_loop(0, CPT, body, 0)
    plsc.subcore_barrier()
    pltpu.sync_copy(acc_sh.at[pl.ds(s * RPT, RPT)],
                    out_hbm.at[c, pl.ds(s * RPT, RPT)])


@functools.partial(
    pl.kernel,
    mesh=_mesh,
    out_type=jax.ShapeDtypeStruct((NC, NPAD, F), jnp.float32),
    scratch_types=[
        pltpu.VMEM((CPP + 8, CHUNK), jnp.int32),
        pltpu.VMEM((CPP, CHUNK), jnp.int32),
        pltpu.VMEM((CHUNK, F), jnp.float32),
        pltpu.VMEM((CHUNK, F), jnp.float32),
        pltpu.VMEM_SHARED((NPAD, F), jnp.float32),
        pltpu.SemaphoreType.DMA,
        pltpu.SemaphoreType.DMA,
    ],
)
def _agg_kernel(h_hbm, src_hbm, dst_hbm, out_hbm, srcv, dstv, buf0, buf1,
                acc_sh, sem0, sem1):
    c = lax.axis_index("c")
    s = lax.axis_index("s")
    w = c * NS + s
    pltpu.sync_copy(h_hbm.at[pl.ds(s * RPT, RPT)], acc_sh.at[pl.ds(s * RPT, RPT)])
    plsc.subcore_barrier()

    bufs = (buf0, buf1)
    sems = (sem0, sem1)
    for ph in range(PHASES):
        pltpu.sync_copy(src_hbm.at[w, pl.ds(ph * CPP, CPP + 8)], srcv)
        pltpu.sync_copy(dst_hbm.at[w, pl.ds(ph * CPP, CPP)], dstv)

        def body(j, carry):
            pltpu.async_copy(h_hbm.at[srcv.at[j]], buf0, sem0).wait()
            pltpu.sync_copy(buf0, acc_sh.at[dstv.at[j]], add=True)
            return carry

        lax.fori_loop(0, CPP, body, 0)
    plsc.subcore_barrier()
    pltpu.sync_copy(acc_sh.at[pl.ds(s * RPT, RPT)],
                    out_hbm.at[c, pl.ds(s * RPT, RPT)])


def _mm1_body(x_ref, w_ref, degp_ref, out_ref, dinv_ref):
    deg = degp_ref[0, :, 0:1] + degp_ref[1, :, 0:1] + 1.0
    dinv = lax.rsqrt(jnp.maximum(deg, 1.0))
    dinv_ref[...] = jnp.broadcast_to(dinv, dinv_ref.shape)
    out_ref[...] = jnp.dot(x_ref[...], w_ref[...],
                           preferred_element_type=jnp.float32) * dinv


def _mm2_body(p_ref, h_ref, dinv8_ref, w_ref, b_ref, out_ref):
    dinv = dinv8_ref[:, 0:1]
    agg = p_ref[0] + p_ref[1] - h_ref[...]
    x2 = jnp.maximum(agg * dinv + b_ref[0:1, :], 0.0)
    out_ref[...] = jnp.dot(x2, w_ref[...],
                           preferred_element_type=jnp.float32) * dinv


def _fin_body(q_ref, h_ref, dinv8_ref, b_ref, out_ref):
    dinv = dinv8_ref[:, 0:1]
    out_ref[...] = (q_ref[0] + q_ref[1] - h_ref[...]) * dinv + b_ref[0:1, :]


_B = 1024
_GRID = (NPAD // _B,)
_bs_rows = pl.BlockSpec((_B, F), lambda i: (i, 0))
_bs_w = pl.BlockSpec((F, F), lambda i: (0, 0))
_bs_deg = pl.BlockSpec((NC, _B, F), lambda i: (0, i, 0))
_bs_part = pl.BlockSpec((NC, _B, F), lambda i: (0, i, 0))
_bs_b = pl.BlockSpec((8, F), lambda i: (0, 0))
_bs_dinv = pl.BlockSpec((_B, 8), lambda i: (i, 0))

_mm1 = pl.pallas_call(
    _mm1_body, grid=_GRID,
    in_specs=[_bs_rows, _bs_w, _bs_deg],
    out_specs=[_bs_rows, _bs_dinv],
    out_shape=[jax.ShapeDtypeStruct((NPAD, F), jnp.float32),
               jax.ShapeDtypeStruct((NPAD, 8), jnp.float32)],
)

_mm2 = pl.pallas_call(
    _mm2_body, grid=_GRID,
    in_specs=[_bs_part, _bs_rows, _bs_dinv, _bs_w, _bs_b],
    out_specs=_bs_rows,
    out_shape=jax.ShapeDtypeStruct((NPAD, F), jnp.float32),
)

_fin = pl.pallas_call(
    _fin_body, grid=_GRID,
    in_specs=[_bs_part, _bs_rows, _bs_dinv, _bs_b],
    out_specs=_bs_rows,
    out_shape=jax.ShapeDtypeStruct((NPAD, F), jnp.float32),
)


def kernel(x, edge_index, W1, b1, W2, b2):
    src = edge_index[0].astype(jnp.int32)
    dst = edge_index[1].astype(jnp.int32)
    npad_e = EPAD - src.shape[0]
    src_p = jnp.concatenate([src, jnp.zeros((npad_e,), jnp.int32)]) \
        .reshape(NW, CPT, CHUNK)
    src_p = jnp.concatenate([src_p, jnp.zeros((NW, 8, CHUNK), jnp.int32)], axis=1)
    dst_p = jnp.concatenate([dst, jnp.full((npad_e,), N_NODES, jnp.int32)]) \
        .reshape(NW, CPT, CHUNK)
    x_p = jnp.zeros((NPAD, F), jnp.float32).at[:N_NODES].set(x)
    b1_p = jnp.zeros((8, F), jnp.float32).at[0].set(b1)
    b2_p = jnp.zeros((8, F), jnp.float32).at[0].set(b2)

    ones_c = jnp.ones((CHUNK, F), jnp.float32)
    zeros_c = jnp.zeros((NPAD, F), jnp.float32)
    degp = _deg_kernel(dst_p, ones_c, zeros_c)
    h1s, dinv8 = _mm1(x_p, W1, degp)
    p = _agg_kernel(h1s, src_p, dst_p)
    h2s = _mm2(p, h1s, dinv8, W2, b1_p)
    q = _agg_kernel(h2s, src_p, dst_p)
    out = _fin(q, h2s, dinv8, b2_p)
    return out[:N_NODES]


# speedup vs baseline: 1.2093x; 1.0005x over previous
import functools

import jax
import jax.numpy as jnp
from jax import lax
from jax.experimental import pallas as pl
from jax.experimental.pallas import tpu as pltpu
from jax.experimental.pallas import tpu_sc as plsc

N_NODES = 10000
F = 128
NC = 2
NS = 16
NW = NC * NS
CHUNK = 128
NPAD = 10240
RPT = NPAD // NS
E_EDGES = 320000
CPT = 80
PHASES = 2
CPP = CPT // PHASES
EPAD = NW * CHUNK * CPT

_mesh = plsc.VectorSubcoreMesh(core_axis_name="c", subcore_axis_name="s")


@functools.partial(
    pl.kernel,
    mesh=_mesh,
    out_type=jax.ShapeDtypeStruct((NC, NPAD, F), jnp.float32),
    scratch_types=[
        pltpu.VMEM((CPT, CHUNK), jnp.int32),
        pltpu.VMEM((CHUNK, F), jnp.float32),
        pltpu.VMEM_SHARED((NPAD, F), jnp.float32),
    ],
)
def _deg_kernel(dst_hbm, ones_hbm, zeros_hbm, out_hbm, idx_v, ones_v, acc_sh):
    c = lax.axis_index("c")
    s = lax.axis_index("s")
    w = c * NS + s
    pltpu.sync_copy(ones_hbm, ones_v)
    pltpu.sync_copy(zeros_hbm.at[pl.ds(s * RPT, RPT)],
                    acc_sh.at[pl.ds(s * RPT, RPT)])
    pltpu.sync_copy(dst_hbm.at[w], idx_v)
    plsc.subcore_barrier()

    def body(j, carry):
        pltpu.sync_copy(ones_v, acc_sh.at[idx_v.at[j]], add=True)
        return carry

    lax.fori_loop(0, CPT, body, 0)
    plsc.subcore_barrier()
    pltpu.sync_copy(acc_sh.at[pl.ds(s * RPT, RPT)],
                    out_hbm.at[c, pl.ds(s * RPT, RPT)])


@functools.partial(
    pl.kernel,
    mesh=_mesh,
    out_type=jax.ShapeDtypeStruct((NC, NPAD, F), jnp.float32),
    scratch_types=[
        pltpu.VMEM((CPP + 8, CHUNK), jnp.int32),
        pltpu.VMEM((CPP, CHUNK), jnp.int32),
        pltpu.VMEM((CHUNK, F), jnp.float32),
        pltpu.VMEM((CHUNK, F), jnp.float32),
        pltpu.VMEM_SHARED((NPAD, F), jnp.float32),
        pltpu.SemaphoreType.DMA,
        pltpu.SemaphoreType.DMA,
    ],
)
def _agg_kernel(h_hbm, src_hbm, dst_hbm, out_hbm, srcv, dstv, buf0, buf1,
                acc_sh, sem0, sem1):
    c = lax.axis_index("c")
    s = lax.axis_index("s")
    w = c * NS + s
    pltpu.sync_copy(h_hbm.at[pl.ds(s * RPT, RPT)], acc_sh.at[pl.ds(s * RPT, RPT)])
    plsc.subcore_barrier()

    bufs = (buf0, buf1)
    sems = (sem0, sem1)
    for ph in range(PHASES):
        pltpu.sync_copy(src_hbm.at[w, pl.ds(ph * CPP, CPP + 8)], srcv)
        pltpu.sync_copy(dst_hbm.at[w, pl.ds(ph * CPP, CPP)], dstv)

        def body(j, carry):
            pltpu.async_copy(h_hbm.at[srcv.at[j]], buf0, sem0).wait()
            pltpu.sync_copy(buf0, acc_sh.at[dstv.at[j]], add=True)
            return carry

        lax.fori_loop(0, CPP, body, 0)
    plsc.subcore_barrier()
    pltpu.sync_copy(acc_sh.at[pl.ds(s * RPT, RPT)],
                    out_hbm.at[c, pl.ds(s * RPT, RPT)])


def _mm1_body(x_ref, w_ref, degp_ref, out_ref, dinv_ref):
    deg = degp_ref[0, :, 0:1] + degp_ref[1, :, 0:1] + 1.0
    dinv = lax.rsqrt(jnp.maximum(deg, 1.0))
    dinv_ref[...] = jnp.broadcast_to(dinv, dinv_ref.shape)
    out_ref[...] = jnp.dot(x_ref[...], w_ref[...],
                           preferred_element_type=jnp.float32) * dinv


def _mm2_body(p_ref, h_ref, dinv8_ref, w_ref, b_ref, out_ref):
    dinv = dinv8_ref[:, 0:1]
    agg = p_ref[0] + p_ref[1] - h_ref[...]
    x2 = jnp.maximum(agg * dinv + b_ref[0:1, :], 0.0)
    out_ref[...] = jnp.dot(x2, w_ref[...],
                           preferred_element_type=jnp.float32) * dinv


def _fin_body(q_ref, h_ref, dinv8_ref, b_ref, out_ref):
    dinv = dinv8_ref[:, 0:1]
    out_ref[...] = (q_ref[0] + q_ref[1] - h_ref[...]) * dinv + b_ref[0:1, :]


_B = 1024
_GRID = (NPAD // _B,)
_bs_rows = pl.BlockSpec((_B, F), lambda i: (i, 0))
_bs_w = pl.BlockSpec((F, F), lambda i: (0, 0))
_bs_deg = pl.BlockSpec((NC, _B, F), lambda i: (0, i, 0))
_bs_part = pl.BlockSpec((NC, _B, F), lambda i: (0, i, 0))
_bs_b = pl.BlockSpec((8, F), lambda i: (0, 0))
_bs_dinv = pl.BlockSpec((_B, 8), lambda i: (i, 0))

_mm1 = pl.pallas_call(
    _mm1_body, grid=_GRID,
    in_specs=[_bs_rows, _bs_w, _bs_deg],
    out_specs=[_bs_rows, _bs_dinv],
    out_shape=[jax.ShapeDtypeStruct((NPAD, F), jnp.float32),
               jax.ShapeDtypeStruct((NPAD, 8), jnp.float32)],
)

_mm2 = pl.pallas_call(
    _mm2_body, grid=_GRID,
    in_specs=[_bs_part, _bs_rows, _bs_dinv, _bs_w, _bs_b],
    out_specs=_bs_rows,
    out_shape=jax.ShapeDtypeStruct((NPAD, F), jnp.float32),
)

_fin = pl.pallas_call(
    _fin_body, grid=_GRID,
    in_specs=[_bs_part, _bs_rows, _bs_dinv, _bs_b],
    out_specs=_bs_rows,
    out_shape=jax.ShapeDtypeStruct((NPAD, F), jnp.float32),
)


def kernel(x, edge_index, W1, b1, W2, b2):
    src = edge_index[0].astype(jnp.int32)
    dst = edge_index[1].astype(jnp.int32)
    npad_e = EPAD - src.shape[0]
    src_p = jnp.concatenate([src, jnp.zeros((npad_e,), jnp.int32)]) \
        .reshape(NW, CPT, CHUNK)
    src_p = jnp.concatenate([src_p, jnp.zeros((NW, 8, CHUNK), jnp.int32)], axis=1)
    pad_dst = N_NODES + (jnp.arange(npad_e, dtype=jnp.int32) % (NPAD - N_NODES))
    dst_p = jnp.concatenate([dst, pad_dst]).reshape(NW, CPT, CHUNK)
    x_p = jnp.zeros((NPAD, F), jnp.float32).at[:N_NODES].set(x)
    b1_p = jnp.zeros((8, F), jnp.float32).at[0].set(b1)
    b2_p = jnp.zeros((8, F), jnp.float32).at[0].set(b2)

    ones_c = jnp.ones((CHUNK, F), jnp.float32)
    zeros_c = jnp.zeros((NPAD, F), jnp.float32)
    degp = _deg_kernel(dst_p, ones_c, zeros_c)
    h1s, dinv8 = _mm1(x_p, W1, degp)
    p = _agg_kernel(h1s, src_p, dst_p)
    h2s = _mm2(p, h1s, dinv8, W2, b1_p)
    q = _agg_kernel(h2s, src_p, dst_p)
    out = _fin(q, h2s, dinv8, b2_p)
    return out[:N_NODES]


# speedup vs baseline: 1.2111x; 1.0015x over previous
import functools

import jax
import jax.numpy as jnp
from jax import lax
from jax.experimental import pallas as pl
from jax.experimental.pallas import tpu as pltpu
from jax.experimental.pallas import tpu_sc as plsc

N_NODES = 10000
F = 128
NC = 2
NS = 16
NW = NC * NS
CHUNK = 128
NPAD = 10240
RPT = NPAD // NS
E_EDGES = 320000
CPT = 80
PHASES = 1
CPP = CPT // PHASES
EPAD = NW * CHUNK * CPT

_mesh = plsc.VectorSubcoreMesh(core_axis_name="c", subcore_axis_name="s")


@functools.partial(
    pl.kernel,
    mesh=_mesh,
    out_type=jax.ShapeDtypeStruct((NC, NPAD, F), jnp.float32),
    scratch_types=[
        pltpu.VMEM((CPT, CHUNK), jnp.int32),
        pltpu.VMEM((CHUNK, F), jnp.float32),
        pltpu.VMEM_SHARED((NPAD, F), jnp.float32),
    ],
)
def _deg_kernel(dst_hbm, ones_hbm, zeros_hbm, out_hbm, idx_v, ones_v, acc_sh):
    c = lax.axis_index("c")
    s = lax.axis_index("s")
    w = c * NS + s
    pltpu.sync_copy(ones_hbm, ones_v)
    pltpu.sync_copy(zeros_hbm.at[pl.ds(s * RPT, RPT)],
                    acc_sh.at[pl.ds(s * RPT, RPT)])
    pltpu.sync_copy(dst_hbm.at[w], idx_v)
    plsc.subcore_barrier()

    def body(j, carry):
        pltpu.sync_copy(ones_v, acc_sh.at[idx_v.at[j]], add=True)
        return carry

    lax.fori_loop(0, CPT, body, 0)
    plsc.subcore_barrier()
    pltpu.sync_copy(acc_sh.at[pl.ds(s * RPT, RPT)],
                    out_hbm.at[c, pl.ds(s * RPT, RPT)])


@functools.partial(
    pl.kernel,
    mesh=_mesh,
    out_type=jax.ShapeDtypeStruct((NC, NPAD, F), jnp.float32),
    scratch_types=[
        pltpu.VMEM((CPP + 8, CHUNK), jnp.int32),
        pltpu.VMEM((CPP, CHUNK), jnp.int32),
        pltpu.VMEM((CHUNK, F), jnp.float32),
        pltpu.VMEM_SHARED((NPAD, F), jnp.float32),
        pltpu.SemaphoreType.DMA,
    ],
)
def _agg_kernel(h_hbm, src_hbm, dst_hbm, out_hbm, srcv, dstv, buf0,
                acc_sh, sem0):
    c = lax.axis_index("c")
    s = lax.axis_index("s")
    w = c * NS + s
    pltpu.sync_copy(h_hbm.at[pl.ds(s * RPT, RPT)], acc_sh.at[pl.ds(s * RPT, RPT)])
    plsc.subcore_barrier()

    for ph in range(PHASES):
        pltpu.sync_copy(src_hbm.at[w, pl.ds(ph * CPP, CPP + 8)], srcv)
        pltpu.sync_copy(dst_hbm.at[w, pl.ds(ph * CPP, CPP)], dstv)

        def body(j, carry):
            pltpu.async_copy(h_hbm.at[srcv.at[j]], buf0, sem0).wait()
            pltpu.sync_copy(buf0, acc_sh.at[dstv.at[j]], add=True)
            return carry

        lax.fori_loop(0, CPP, body, 0)
    plsc.subcore_barrier()
    pltpu.sync_copy(acc_sh.at[pl.ds(s * RPT, RPT)],
                    out_hbm.at[c, pl.ds(s * RPT, RPT)])


def _mm1_body(x_ref, w_ref, degp_ref, out_ref, dinv_ref):
    deg = degp_ref[0, :, 0:1] + degp_ref[1, :, 0:1] + 1.0
    dinv = lax.rsqrt(jnp.maximum(deg, 1.0))
    dinv_ref[...] = jnp.broadcast_to(dinv, dinv_ref.shape)
    out_ref[...] = jnp.dot(x_ref[...], w_ref[...],
                           preferred_element_type=jnp.float32) * dinv


def _mm2_body(p_ref, h_ref, dinv8_ref, w_ref, b_ref, out_ref):
    dinv = dinv8_ref[:, 0:1]
    agg = p_ref[0] + p_ref[1] - h_ref[...]
    x2 = jnp.maximum(agg * dinv + b_ref[0:1, :], 0.0)
    out_ref[...] = jnp.dot(x2, w_ref[...],
                           preferred_element_type=jnp.float32) * dinv


def _fin_body(q_ref, h_ref, dinv8_ref, b_ref, out_ref):
    dinv = dinv8_ref[:, 0:1]
    out_ref[...] = (q_ref[0] + q_ref[1] - h_ref[...]) * dinv + b_ref[0:1, :]


_B = 1024
_GRID = (NPAD // _B,)
_bs_rows = pl.BlockSpec((_B, F), lambda i: (i, 0))
_bs_w = pl.BlockSpec((F, F), lambda i: (0, 0))
_bs_deg = pl.BlockSpec((NC, _B, F), lambda i: (0, i, 0))
_bs_part = pl.BlockSpec((NC, _B, F), lambda i: (0, i, 0))
_bs_b = pl.BlockSpec((8, F), lambda i: (0, 0))
_bs_dinv = pl.BlockSpec((_B, 8), lambda i: (i, 0))

_mm1 = pl.pallas_call(
    _mm1_body, grid=_GRID,
    in_specs=[_bs_rows, _bs_w, _bs_deg],
    out_specs=[_bs_rows, _bs_dinv],
    out_shape=[jax.ShapeDtypeStruct((NPAD, F), jnp.float32),
               jax.ShapeDtypeStruct((NPAD, 8), jnp.float32)],
)

_mm2 = pl.pallas_call(
    _mm2_body, grid=_GRID,
    in_specs=[_bs_part, _bs_rows, _bs_dinv, _bs_w, _bs_b],
    out_specs=_bs_rows,
    out_shape=jax.ShapeDtypeStruct((NPAD, F), jnp.float32),
)

_fin = pl.pallas_call(
    _fin_body, grid=_GRID,
    in_specs=[_bs_part, _bs_rows, _bs_dinv, _bs_b],
    out_specs=_bs_rows,
    out_shape=jax.ShapeDtypeStruct((NPAD, F), jnp.float32),
)


def kernel(x, edge_index, W1, b1, W2, b2):
    src = edge_index[0].astype(jnp.int32)
    dst = edge_index[1].astype(jnp.int32)
    npad_e = EPAD - src.shape[0]
    src_p = jnp.concatenate([src, jnp.zeros((npad_e,), jnp.int32)]) \
        .reshape(NW, CPT, CHUNK)
    src_p = jnp.concatenate([src_p, jnp.zeros((NW, 8, CHUNK), jnp.int32)], axis=1)
    pad_dst = N_NODES + (jnp.arange(npad_e, dtype=jnp.int32) % (NPAD - N_NODES))
    dst_p = jnp.concatenate([dst, pad_dst]).reshape(NW, CPT, CHUNK)
    x_p = jnp.zeros((NPAD, F), jnp.float32).at[:N_NODES].set(x)
    b1_p = jnp.zeros((8, F), jnp.float32).at[0].set(b1)
    b2_p = jnp.zeros((8, F), jnp.float32).at[0].set(b2)

    ones_c = jnp.ones((CHUNK, F), jnp.float32)
    zeros_c = jnp.zeros((NPAD, F), jnp.float32)
    degp = _deg_kernel(dst_p, ones_c, zeros_c)
    h1s, dinv8 = _mm1(x_p, W1, degp)
    p = _agg_kernel(h1s, src_p, dst_p)
    h2s = _mm2(p, h1s, dinv8, W2, b1_p)
    q = _agg_kernel(h2s, src_p, dst_p)
    out = _fin(q, h2s, dinv8, b2_p)
    return out[:N_NODES]


# speedup vs baseline: 2.7100x; 2.2376x over previous
import functools

import jax
import jax.numpy as jnp
from jax import lax
from jax.experimental import pallas as pl
from jax.experimental.pallas import tpu as pltpu
from jax.experimental.pallas import tpu_sc as plsc

N_NODES = 10000
F = 128
NC = 2
NS = 16
NW = NC * NS
CHUNK = 128
NPAD = 10240
RPT = NPAD // NS
E_EDGES = 320000
CPT = 79
EPAD = NW * CHUNK * CPT

_mesh = plsc.VectorSubcoreMesh(core_axis_name="c", subcore_axis_name="s")


@functools.partial(
    pl.kernel,
    mesh=_mesh,
    out_type=jax.ShapeDtypeStruct((NC, NPAD, F), jnp.float32),
    scratch_types=[
        pltpu.VMEM((CPT + 1, CHUNK), jnp.int32),
        pltpu.VMEM((CHUNK, F), jnp.float32),
        pltpu.VMEM_SHARED((NPAD, F), jnp.float32),
    ],
)
def _deg_kernel(dst_hbm, ones_hbm, zeros_hbm, out_hbm, idx_v, ones_v, acc_sh):
    c = lax.axis_index("c")
    s = lax.axis_index("s")
    w = c * NS + s
    pltpu.sync_copy(ones_hbm, ones_v)
    pltpu.sync_copy(zeros_hbm.at[pl.ds(s * RPT, RPT)],
                    acc_sh.at[pl.ds(s * RPT, RPT)])
    pltpu.sync_copy(dst_hbm.at[w], idx_v)
    plsc.subcore_barrier()

    def body(j, carry):
        pltpu.sync_copy(ones_v, acc_sh.at[idx_v.at[j]], add=True)
        return carry

    lax.fori_loop(0, CPT, body, 0)
    plsc.subcore_barrier()
    pltpu.sync_copy(acc_sh.at[pl.ds(s * RPT, RPT)],
                    out_hbm.at[c, pl.ds(s * RPT, RPT)])


@functools.partial(
    pl.kernel,
    mesh=_mesh,
    out_type=jax.ShapeDtypeStruct((NC, NPAD, F), jnp.float32),
    scratch_types=[
        pltpu.VMEM((CPT + 1, CHUNK), jnp.int32),
        pltpu.VMEM((CPT + 1, CHUNK), jnp.int32),
        pltpu.VMEM((CHUNK, F), jnp.float32),
        pltpu.VMEM_SHARED((NPAD, F), jnp.float32),
        pltpu.SemaphoreType.DMA,
    ],
)
def _agg_kernel(h_hbm, src_hbm, dst_hbm, out_hbm, srcv, dstv, buf0,
                acc_sh, sem0):
    c = lax.axis_index("c")
    s = lax.axis_index("s")
    w = c * NS + s
    pltpu.sync_copy(h_hbm.at[pl.ds(s * RPT, RPT)], acc_sh.at[pl.ds(s * RPT, RPT)])
    pltpu.sync_copy(src_hbm.at[w], srcv)
    pltpu.sync_copy(dst_hbm.at[w], dstv)
    plsc.subcore_barrier()

    def body(j, carry):
        pltpu.async_copy(h_hbm.at[srcv.at[j]], buf0, sem0).wait()
        pltpu.sync_copy(buf0, acc_sh.at[dstv.at[j]], add=True)
        return carry

    lax.fori_loop(0, CPT, body, 0)
    plsc.subcore_barrier()
    pltpu.sync_copy(acc_sh.at[pl.ds(s * RPT, RPT)],
                    out_hbm.at[c, pl.ds(s * RPT, RPT)])


def _mm1_body(x_ref, w_ref, degp_ref, out_ref, dinv_ref):
    deg = degp_ref[0, :, 0:1] + degp_ref[1, :, 0:1] + 1.0
    dinv = lax.rsqrt(jnp.maximum(deg, 1.0))
    dinv_ref[...] = jnp.broadcast_to(dinv, dinv_ref.shape)
    out_ref[...] = jnp.dot(x_ref[...], w_ref[...],
                           preferred_element_type=jnp.float32) * dinv


def _mm2_body(p_ref, h_ref, dinv8_ref, w_ref, b_ref, out_ref):
    dinv = dinv8_ref[:, 0:1]
    agg = p_ref[0] + p_ref[1] - h_ref[...]
    x2 = jnp.maximum(agg * dinv + b_ref[0:1, :], 0.0)
    out_ref[...] = jnp.dot(x2, w_ref[...],
                           preferred_element_type=jnp.float32) * dinv


def _fin_body(q_ref, h_ref, dinv8_ref, b_ref, out_ref):
    dinv = dinv8_ref[:, 0:1]
    out_ref[...] = (q_ref[0] + q_ref[1] - h_ref[...]) * dinv + b_ref[0:1, :]


_B = 1024
_GRID = (NPAD // _B,)
_bs_rows = pl.BlockSpec((_B, F), lambda i: (i, 0))
_bs_w = pl.BlockSpec((F, F), lambda i: (0, 0))
_bs_deg = pl.BlockSpec((NC, _B, F), lambda i: (0, i, 0))
_bs_part = pl.BlockSpec((NC, _B, F), lambda i: (0, i, 0))
_bs_b = pl.BlockSpec((8, F), lambda i: (0, 0))
_bs_dinv = pl.BlockSpec((_B, 8), lambda i: (i, 0))

_mm1 = pl.pallas_call(
    _mm1_body, grid=_GRID,
    in_specs=[_bs_rows, _bs_w, _bs_deg],
    out_specs=[_bs_rows, _bs_dinv],
    out_shape=[jax.ShapeDtypeStruct((NPAD, F), jnp.float32),
               jax.ShapeDtypeStruct((NPAD, 8), jnp.float32)],
)

_mm2 = pl.pallas_call(
    _mm2_body, grid=_GRID,
    in_specs=[_bs_part, _bs_rows, _bs_dinv, _bs_w, _bs_b],
    out_specs=_bs_rows,
    out_shape=jax.ShapeDtypeStruct((NPAD, F), jnp.float32),
)

_fin = pl.pallas_call(
    _fin_body, grid=_GRID,
    in_specs=[_bs_part, _bs_rows, _bs_dinv, _bs_b],
    out_specs=_bs_rows,
    out_shape=jax.ShapeDtypeStruct((NPAD, F), jnp.float32),
)


def kernel(x, edge_index, W1, b1, W2, b2):
    src = edge_index[0].astype(jnp.int32)
    dst = edge_index[1].astype(jnp.int32)
    npad_e = EPAD - src.shape[0]
    pad_src = jnp.arange(npad_e, dtype=jnp.int32) % N_NODES
    pad_dst = N_NODES + (jnp.arange(npad_e, dtype=jnp.int32) % (NPAD - N_NODES))
    src_p = jnp.concatenate([src, pad_src]).reshape(NW, CPT, CHUNK)
    dst_p = jnp.concatenate([dst, pad_dst]).reshape(NW, CPT, CHUNK)
    filler = jnp.zeros((NW, 1, CHUNK), jnp.int32)
    src_p = jnp.concatenate([src_p, filler], axis=1)
    dst_p = jnp.concatenate([dst_p, filler], axis=1)
    x_p = jnp.zeros((NPAD, F), jnp.float32).at[:N_NODES].set(x)
    b1_p = jnp.zeros((8, F), jnp.float32).at[0].set(b1)
    b2_p = jnp.zeros((8, F), jnp.float32).at[0].set(b2)

    ones_c = jnp.ones((CHUNK, F), jnp.float32)
    zeros_c = jnp.zeros((NPAD, F), jnp.float32)
    degp = _deg_kernel(dst_p, ones_c, zeros_c)
    h1s, dinv8 = _mm1(x_p, W1, degp)
    p = _agg_kernel(h1s, src_p, dst_p)
    h2s = _mm2(p, h1s, dinv8, W2, b1_p)
    q = _agg_kernel(h2s, src_p, dst_p)
    out = _fin(q, h2s, dinv8, b2_p)
    return out[:N_NODES]


# speedup vs baseline: 2.9915x; 1.1039x over previous
import functools

import jax
import jax.numpy as jnp
from jax import lax
from jax.experimental import pallas as pl
from jax.experimental.pallas import tpu as pltpu
from jax.experimental.pallas import tpu_sc as plsc

N_NODES = 10000
F = 128
NC = 2
NS = 16
NW = NC * NS
CHUNK = 128
NPAD = 10240
RPT = NPAD // NS
E_EDGES = 320000
CPT = 79
EPAD = NW * CHUNK * CPT

_mesh = plsc.VectorSubcoreMesh(core_axis_name="c", subcore_axis_name="s")


@functools.partial(
    pl.kernel,
    mesh=_mesh,
    out_type=jax.ShapeDtypeStruct((NC, NPAD, F), jnp.float32),
    scratch_types=[
        pltpu.VMEM((CPT + 1, CHUNK), jnp.int32),
        pltpu.VMEM((CHUNK, F), jnp.float32),
        pltpu.VMEM_SHARED((NPAD, F), jnp.float32),
    ],
)
def _deg_kernel(dst_hbm, ones_hbm, zeros_hbm, out_hbm, idx_v, ones_v, acc_sh):
    c = lax.axis_index("c")
    s = lax.axis_index("s")
    w = c * NS + s
    pltpu.sync_copy(ones_hbm, ones_v)
    pltpu.sync_copy(zeros_hbm.at[pl.ds(s * RPT, RPT)],
                    acc_sh.at[pl.ds(s * RPT, RPT)])
    pltpu.sync_copy(dst_hbm.at[w], idx_v)
    plsc.subcore_barrier()

    def body(j, carry):
        pltpu.sync_copy(ones_v, acc_sh.at[idx_v.at[j]], add=True)
        return carry

    lax.fori_loop(0, CPT, body, 0)
    plsc.subcore_barrier()
    pltpu.sync_copy(acc_sh.at[pl.ds(s * RPT, RPT)],
                    out_hbm.at[c, pl.ds(s * RPT, RPT)])


@functools.partial(
    pl.kernel,
    mesh=_mesh,
    out_type=jax.ShapeDtypeStruct((NC, NPAD, F), jnp.float32),
    scratch_types=[
        pltpu.VMEM((40, CHUNK), jnp.int32),
        pltpu.VMEM((40, CHUNK), jnp.int32),
        pltpu.VMEM((CHUNK, F), jnp.float32),
        pltpu.VMEM((CHUNK, F), jnp.float32),
        pltpu.VMEM_SHARED((NPAD, F), jnp.float32),
        pltpu.SemaphoreType.DMA,
        pltpu.SemaphoreType.DMA,
    ],
)
def _agg_kernel(h_hbm, src_hbm, dst_hbm, out_hbm, srcv, dstv, buf0, buf1,
                acc_sh, sem0, sem1):
    c = lax.axis_index("c")
    s = lax.axis_index("s")
    w = c * NS + s
    pltpu.sync_copy(h_hbm.at[pl.ds(s * RPT, RPT)], acc_sh.at[pl.ds(s * RPT, RPT)])
    plsc.subcore_barrier()

    for ph, n in ((0, 40), (1, CPT - 40)):
        pltpu.sync_copy(src_hbm.at[w, pl.ds(ph * 40, 40)], srcv)
        pltpu.sync_copy(dst_hbm.at[w, pl.ds(ph * 40, 40)], dstv)

        def body(it, carry):
            jj = it * 2
            cpa = pltpu.async_copy(h_hbm.at[srcv.at[jj]], buf0, sem0)
            cpb = pltpu.async_copy(h_hbm.at[srcv.at[jj + 1]], buf1, sem1)
            cpa.wait()
            pltpu.sync_copy(buf0, acc_sh.at[dstv.at[jj]], add=True)
            cpb.wait()
            pltpu.sync_copy(buf1, acc_sh.at[dstv.at[jj + 1]], add=True)
            return carry

        lax.fori_loop(0, n // 2, body, 0)
        if n % 2:
            pltpu.async_copy(h_hbm.at[srcv.at[n - 1]], buf0, sem0).wait()
            pltpu.sync_copy(buf0, acc_sh.at[dstv.at[n - 1]], add=True)
    plsc.subcore_barrier()
    pltpu.sync_copy(acc_sh.at[pl.ds(s * RPT, RPT)],
                    out_hbm.at[c, pl.ds(s * RPT, RPT)])


def _mm1_body(x_ref, w_ref, degp_ref, out_ref, dinv_ref):
    deg = degp_ref[0, :, 0:1] + degp_ref[1, :, 0:1] + 1.0
    dinv = lax.rsqrt(jnp.maximum(deg, 1.0))
    dinv_ref[...] = jnp.broadcast_to(dinv, dinv_ref.shape)
    out_ref[...] = jnp.dot(x_ref[...], w_ref[...],
                           preferred_element_type=jnp.float32) * dinv


def _mm2_body(p_ref, h_ref, dinv8_ref, w_ref, b_ref, out_ref):
    dinv = dinv8_ref[:, 0:1]
    agg = p_ref[0] + p_ref[1] - h_ref[...]
    x2 = jnp.maximum(agg * dinv + b_ref[0:1, :], 0.0)
    out_ref[...] = jnp.dot(x2, w_ref[...],
                           preferred_element_type=jnp.float32) * dinv


def _fin_body(q_ref, h_ref, dinv8_ref, b_ref, out_ref):
    dinv = dinv8_ref[:, 0:1]
    out_ref[...] = (q_ref[0] + q_ref[1] - h_ref[...]) * dinv + b_ref[0:1, :]


_B = 1024
_GRID = (NPAD // _B,)
_bs_rows = pl.BlockSpec((_B, F), lambda i: (i, 0))
_bs_w = pl.BlockSpec((F, F), lambda i: (0, 0))
_bs_deg = pl.BlockSpec((NC, _B, F), lambda i: (0, i, 0))
_bs_part = pl.BlockSpec((NC, _B, F), lambda i: (0, i, 0))
_bs_b = pl.BlockSpec((8, F), lambda i: (0, 0))
_bs_dinv = pl.BlockSpec((_B, 8), lambda i: (i, 0))

_mm1 = pl.pallas_call(
    _mm1_body, grid=_GRID,
    in_specs=[_bs_rows, _bs_w, _bs_deg],
    out_specs=[_bs_rows, _bs_dinv],
    out_shape=[jax.ShapeDtypeStruct((NPAD, F), jnp.float32),
               jax.ShapeDtypeStruct((NPAD, 8), jnp.float32)],
)

_mm2 = pl.pallas_call(
    _mm2_body, grid=_GRID,
    in_specs=[_bs_part, _bs_rows, _bs_dinv, _bs_w, _bs_b],
    out_specs=_bs_rows,
    out_shape=jax.ShapeDtypeStruct((NPAD, F), jnp.float32),
)

_fin = pl.pallas_call(
    _fin_body, grid=_GRID,
    in_specs=[_bs_part, _bs_rows, _bs_dinv, _bs_b],
    out_specs=_bs_rows,
    out_shape=jax.ShapeDtypeStruct((NPAD, F), jnp.float32),
)


def kernel(x, edge_index, W1, b1, W2, b2):
    src = edge_index[0].astype(jnp.int32)
    dst = edge_index[1].astype(jnp.int32)
    npad_e = EPAD - src.shape[0]
    pad_src = jnp.arange(npad_e, dtype=jnp.int32) % N_NODES
    pad_dst = N_NODES + (jnp.arange(npad_e, dtype=jnp.int32) % (NPAD - N_NODES))
    src_p = jnp.concatenate([src, pad_src]).reshape(NW, CPT, CHUNK)
    dst_p = jnp.concatenate([dst, pad_dst]).reshape(NW, CPT, CHUNK)
    filler = jnp.zeros((NW, 1, CHUNK), jnp.int32)
    src_p = jnp.concatenate([src_p, filler], axis=1)
    dst_p = jnp.concatenate([dst_p, filler], axis=1)
    x_p = jnp.zeros((NPAD, F), jnp.float32).at[:N_NODES].set(x)
    b1_p = jnp.zeros((8, F), jnp.float32).at[0].set(b1)
    b2_p = jnp.zeros((8, F), jnp.float32).at[0].set(b2)

    ones_c = jnp.ones((CHUNK, F), jnp.float32)
    zeros_c = jnp.zeros((NPAD, F), jnp.float32)
    degp = _deg_kernel(dst_p, ones_c, zeros_c)
    h1s, dinv8 = _mm1(x_p, W1, degp)
    p = _agg_kernel(h1s, src_p, dst_p)
    h2s = _mm2(p, h1s, dinv8, W2, b1_p)
    q = _agg_kernel(h2s, src_p, dst_p)
    out = _fin(q, h2s, dinv8, b2_p)
    return out[:N_NODES]
